# baseline (speedup 1.0000x reference)
import jax
import jax.numpy as jnp
from jax import lax
from jax.experimental import pallas as pl
from jax.experimental.pallas import tpu as pltpu
from jax.experimental.pallas import tpu_sc as plsc

N_ROWS = 32
ROWS = 16384
COLS = 128
K_SC = 4

CH = 4096
PER_ROW = ROWS // CH
NCH = (N_ROWS - K_SC) * PER_ROW
NBUF = 16
W = 8
K = NBUF - W
NGRP = -(-NCH // NBUF)


def _rd(x_hbm, buf, sem, i):
    r = K_SC + i // PER_ROW
    sl = pl.ds((i % PER_ROW) * CH, CH)
    return pltpu.make_async_copy(x_hbm.at[r, sl], buf, sem)


def _wr(out_hbm, buf, sem, i):
    r = K_SC + i // PER_ROW
    sl = pl.ds((i % PER_ROW) * CH, CH)
    return pltpu.make_async_copy(buf, out_hbm.at[r, sl], sem)


def _tc_body(x_hbm, out_hbm, *scratch):
    bufs = scratch[:NBUF]
    rsems = scratch[NBUF:2 * NBUF]
    wsems = scratch[2 * NBUF:]

    for j in range(K):
        _rd(x_hbm, bufs[j], rsems[j], j).start()

    def body(g, carry):
        for b in range(NBUF):
            i = g * NBUF + b

            @pl.when(i < NCH)
            def _():
                _rd(x_hbm, bufs[b], rsems[b], i).wait()
                _wr(out_hbm, bufs[b], wsems[b], i).start()

            bw = (b - W) % NBUF

            @pl.when(i >= W)
            def _():
                _wr(out_hbm, bufs[bw], wsems[bw], i - W).wait()

            br = (b + K) % NBUF

            @pl.when(i + K < NCH)
            def _():
                _rd(x_hbm, bufs[br], rsems[br], i + K).start()
        return carry

    lax.fori_loop(0, NGRP, body, 0)
    for i in range(NGRP * NBUF - W, NCH):
        b = i % NBUF
        _wr(out_hbm, bufs[b], wsems[b], i).wait()


_tc_pass_through = pl.pallas_call(
    _tc_body,
    out_shape=jax.ShapeDtypeStruct((N_ROWS, ROWS, COLS), jnp.float32),
    in_specs=[pl.BlockSpec(memory_space=pltpu.MemorySpace.HBM)],
    out_specs=pl.BlockSpec(memory_space=pltpu.MemorySpace.HBM),
    scratch_shapes=(
        [pltpu.VMEM((CH, COLS), jnp.float32) for _ in range(NBUF)]
        + [pltpu.SemaphoreType.DMA for _ in range(2 * NBUF)]
    ),
)


SPW = N_ROWS // K_SC
STRIPE = ROWS // SPW
SC_CH = 256
SC_NCH = STRIPE // SC_CH


def _sc_body(x_hbm, src_hbm, out_hbm, b0, b1, r0, r1, w0, w1):
    c = lax.axis_index("c")
    s = lax.axis_index("s")
    w = s * 2 + c
    r = w // SPW
    off = (w % SPW) * STRIPE
    bufs = (b0, b1)
    rsems = (r0, r1)
    wsems = (w1, w0)[::-1]

    def rd(ci, slot):
        sl = pl.ds(off + ci * SC_CH, SC_CH)

        @pl.when(r == 0)
        def _():
            pltpu.make_async_copy(src_hbm.at[sl], bufs[slot],
                                  rsems[slot]).start()

        @pl.when(r != 0)
        def _():
            pltpu.make_async_copy(x_hbm.at[r, sl], bufs[slot],
                                  rsems[slot]).start()

    def rd_wait(ci, slot):
        sl = pl.ds(off + ci * SC_CH, SC_CH)
        pltpu.make_async_copy(x_hbm.at[r, sl], bufs[slot],
                              rsems[slot]).wait()

    def wrc(ci, slot):
        sl = pl.ds(off + ci * SC_CH, SC_CH)
        return pltpu.make_async_copy(bufs[slot], out_hbm.at[r, sl],
                                     wsems[slot])

    rd(0, 0)

    def body(g, carry):
        for gi in range(2):
            ci = g * 2 + gi
            slot = gi
            nslot = (gi + 1) % 2
            rd_wait(ci, slot)
            wrc(ci, slot).start()

            @pl.when(ci >= 1)
            def _():
                wrc(ci - 1, nslot).wait()

            @pl.when(ci + 1 < SC_NCH)
            def _():
                rd(ci + 1, nslot)
        return carry

    lax.fori_loop(0, SC_NCH // 2, body, 0)
    wrc(SC_NCH - 1, (SC_NCH - 1) % 2).wait()


_sc_rows = pl.kernel(
    _sc_body,
    out_type=(),
    mesh=plsc.VectorSubcoreMesh(core_axis_name="c", subcore_axis_name="s"),
    scratch_types=(
        [pltpu.VMEM((SC_CH, COLS), jnp.float32) for _ in range(2)]
        + [pltpu.SemaphoreType.DMA for _ in range(4)]
    ),
)


def kernel(x, src):
    out = _tc_pass_through(x)
    ref = jax.new_ref(out)
    _sc_rows(x, src, ref)
    return ref[...]
